# baseline (device time: 51983 ns/iter reference)
import jax
import jax.numpy as jnp
from jax import lax
from jax.experimental import pallas as pl
from jax.experimental.pallas import tpu as pltpu

N_DEV = 4
EPS = 1e-5


def kernel(x, gamma, beta):
    m, n_loc = x.shape
    n_glob = n_loc * N_DEV

    def body(x_ref, g_ref, b_ref, o_ref, comm_ref, send_sems, recv_sems):
        my = lax.axis_index("i")
        left = lax.rem(my + (N_DEV - 1), N_DEV)
        right = lax.rem(my + 1, N_DEV)

        barrier_sem = pltpu.get_barrier_semaphore()
        for nbr in (left, right):
            pl.semaphore_signal(
                barrier_sem, inc=1,
                device_id=(nbr,), device_id_type=pl.DeviceIdType.MESH,
            )
        pl.semaphore_wait(barrier_sem, 2)

        xv = x_ref[:, :]
        comm_ref[0, :, 0:1] = jnp.sum(xv, axis=1, keepdims=True)
        comm_ref[0, :, 1:2] = jnp.sum(xv * xv, axis=1, keepdims=True)

        for h in range(N_DEV - 1):
            rdma = pltpu.make_async_remote_copy(
                src_ref=comm_ref.at[h],
                dst_ref=comm_ref.at[h + 1],
                send_sem=send_sems.at[h],
                recv_sem=recv_sems.at[h],
                device_id=(right,),
                device_id_type=pl.DeviceIdType.MESH,
            )
            rdma.start()
            rdma.wait()

        tot = (comm_ref[0, :, :] + comm_ref[1, :, :]
               + comm_ref[2, :, :] + comm_ref[3, :, :])
        mean = tot[:, 0:1] / n_glob
        var = tot[:, 1:2] / n_glob - mean * mean
        inv = lax.rsqrt(var + EPS)
        o_ref[:, :] = g_ref[:, :] * (xv - mean) * inv + b_ref[:, :]

    return pl.pallas_call(
        body,
        out_shape=jax.ShapeDtypeStruct((m, n_loc), jnp.float32),
        in_specs=[
            pl.BlockSpec(memory_space=pltpu.VMEM),
            pl.BlockSpec(memory_space=pltpu.VMEM),
            pl.BlockSpec(memory_space=pltpu.VMEM),
        ],
        out_specs=pl.BlockSpec(memory_space=pltpu.VMEM),
        scratch_shapes=[
            pltpu.VMEM((N_DEV, m, 2), jnp.float32),
            pltpu.SemaphoreType.DMA((N_DEV - 1,)),
            pltpu.SemaphoreType.DMA((N_DEV - 1,)),
        ],
        compiler_params=pltpu.CompilerParams(collective_id=0),
    )(x, gamma.reshape(1, n_loc), beta.reshape(1, n_loc))


# device time: 17878 ns/iter; 2.9077x vs baseline; 2.9077x over previous
import jax
import jax.numpy as jnp
from jax import lax
from jax.experimental import pallas as pl
from jax.experimental.pallas import tpu as pltpu

N_DEV = 4
EPS = 1e-5


def kernel(x, gamma, beta):
    m, n_loc = x.shape
    n_glob = n_loc * N_DEV

    def body(x_ref, g_ref, b_ref, o_ref, comm_ref, send_sems, recv_sems):
        my = lax.axis_index("i")

        barrier_sem = pltpu.get_barrier_semaphore()
        for d in range(1, N_DEV):
            pl.semaphore_signal(
                barrier_sem, inc=1,
                device_id=(lax.rem(my + d, N_DEV),),
                device_id_type=pl.DeviceIdType.MESH,
            )
        pl.semaphore_wait(barrier_sem, N_DEV - 1)

        xv = x_ref[:, :]
        ones_row = jnp.ones((1, n_loc), jnp.float32)
        nt = (((1,), (1,)), ((), ()))
        comm_ref[0, 0:1, :] = lax.dot_general(
            ones_row, xv, nt, preferred_element_type=jnp.float32)
        comm_ref[0, 1:2, :] = lax.dot_general(
            ones_row, xv * xv, nt, preferred_element_type=jnp.float32)

        rdmas = []
        for d in range(1, N_DEV):
            rdma = pltpu.make_async_remote_copy(
                src_ref=comm_ref.at[0],
                dst_ref=comm_ref.at[d],
                send_sem=send_sems.at[d - 1],
                recv_sem=recv_sems.at[d - 1],
                device_id=(lax.rem(my + d, N_DEV),),
                device_id_type=pl.DeviceIdType.MESH,
            )
            rdma.start()
            rdmas.append(rdma)
        for rdma in rdmas:
            rdma.wait()

        tot = (comm_ref[0, :, :] + comm_ref[1, :, :]
               + comm_ref[2, :, :] + comm_ref[3, :, :])
        tot_c = tot.T
        mean = tot_c[:, 0:1] / n_glob
        var = tot_c[:, 1:2] / n_glob - mean * mean
        inv = lax.rsqrt(var + EPS)
        o_ref[:, :] = g_ref[:, :] * (xv - mean) * inv + b_ref[:, :]

    return pl.pallas_call(
        body,
        out_shape=jax.ShapeDtypeStruct((m, n_loc), jnp.float32),
        in_specs=[
            pl.BlockSpec(memory_space=pltpu.VMEM),
            pl.BlockSpec(memory_space=pltpu.VMEM),
            pl.BlockSpec(memory_space=pltpu.VMEM),
        ],
        out_specs=pl.BlockSpec(memory_space=pltpu.VMEM),
        scratch_shapes=[
            pltpu.VMEM((N_DEV, 2, m), jnp.float32),
            pltpu.SemaphoreType.DMA((N_DEV - 1,)),
            pltpu.SemaphoreType.DMA((N_DEV - 1,)),
        ],
        compiler_params=pltpu.CompilerParams(collective_id=0),
    )(x, gamma.reshape(1, n_loc), beta.reshape(1, n_loc))
